# Initial kernel scaffold; baseline (speedup 1.0000x reference)
import jax
import jax.numpy as jnp
from jax.experimental import pallas as pl


def kernel(x, edge_index, edge_feats, Wl_0_0, bl_0_0, Wr_0_0, Wl_0_1, bl_0_1, Wr_0_1, Wl_1_0, bl_1_0, Wr_1_0, Wl_1_1, bl_1_1, Wr_1_1):
    raise NotImplementedError("write your pallas kernel here")


# speedup vs baseline: 7.2300x; 7.2300x over previous
import jax
import jax.numpy as jnp
from jax import lax
from jax.experimental import pallas as pl
from jax.experimental.pallas import tpu as pltpu
from jax.experimental.pallas import tpu_sc as plsc

N = 10000
E = 320000
D = 128
NC = 2
NS = 16
NW = NC * NS
EW = E // NW
C = 125
NCHUNK = EW // C
G = 16
NG = NCHUNK // G
NP = 10240
RPT = NP // NS
CW = 128

_mesh = plsc.VectorSubcoreMesh(core_axis_name="c", subcore_axis_name="s")


def _sc_agg_body(h, srcs, dsts, zrows, out_p, srcv, dstv, rowsv, acc, sem):
  cid = lax.axis_index("c")
  sid = lax.axis_index("s")
  wid = cid * NS + sid
  pltpu.sync_copy(zrows.at[pl.ds(sid * RPT, RPT)], acc.at[pl.ds(sid * RPT, RPT)])
  plsc.subcore_barrier()

  def group(g, carry):
    pltpu.sync_copy(srcs.at[pl.ds(wid * NCHUNK + g * G, G)], srcv)
    pltpu.sync_copy(dsts.at[pl.ds(wid * NCHUNK + g * G, G)], dstv)

    def chunk(j, c2):
      pltpu.async_copy(h.at[srcv.at[j]], rowsv, sem).wait()
      pltpu.sync_copy(rowsv, acc.at[dstv.at[j]], add=True)
      return c2

    return lax.fori_loop(0, G, chunk, carry)

  lax.fori_loop(0, NG, group, 0)
  plsc.subcore_barrier()
  pltpu.sync_copy(acc.at[pl.ds(sid * RPT, RPT)],
                  out_p.at[cid, pl.ds(sid * RPT, RPT)])


_sc_agg = pl.kernel(
    _sc_agg_body,
    out_type=jax.ShapeDtypeStruct((NC, NP, D), jnp.float32),
    mesh=_mesh,
    scratch_types=[
        pltpu.VMEM((G, C), jnp.int32),
        pltpu.VMEM((G, C), jnp.int32),
        pltpu.VMEM((C, D), jnp.float32),
        pltpu.VMEM_SHARED((NP, D), jnp.float32),
        pltpu.SemaphoreType.DMA,
    ],
)


def _sc_count_body(dsts, zcnt, ones, out_c, dstv, onesv, cacc):
  cid = lax.axis_index("c")
  sid = lax.axis_index("s")
  wid = cid * NS + sid
  pltpu.sync_copy(ones, onesv)
  pltpu.sync_copy(zcnt.at[pl.ds(sid * RPT, RPT)],
                  cacc.at[pl.ds(sid * RPT, RPT)])
  plsc.subcore_barrier()

  def group(g, carry):
    pltpu.sync_copy(dsts.at[pl.ds(wid * NCHUNK + g * G, G)], dstv)

    def chunk(j, c2):
      pltpu.sync_copy(onesv, cacc.at[dstv.at[j]], add=True)
      return c2

    return lax.fori_loop(0, G, chunk, carry)

  lax.fori_loop(0, NG, group, 0)
  plsc.subcore_barrier()
  pltpu.sync_copy(cacc.at[pl.ds(sid * RPT, RPT)],
                  out_c.at[cid, pl.ds(sid * RPT, RPT)])


_sc_count = pl.kernel(
    _sc_count_body,
    out_type=jax.ShapeDtypeStruct((NC, NP, CW), jnp.float32),
    mesh=_mesh,
    scratch_types=[
        pltpu.VMEM((G, C), jnp.int32),
        pltpu.VMEM((C, CW), jnp.float32),
        pltpu.VMEM_SHARED((NP, CW), jnp.float32),
    ],
)

_R = 400


def _make_tc_layer(act):
  def body(p0, p1, c0, c1, h, wl, wr, bias, out):
    cnt = c0[:, 0:1] + c1[:, 0:1]
    inv = 1.0 / jnp.maximum(cnt, 1.0)
    agg = (p0[...] + p1[...]) * inv
    y = (jnp.dot(agg, wl[...], preferred_element_type=jnp.float32)
         + jnp.dot(h[...], wr[...], preferred_element_type=jnp.float32)
         + bias[...])
    if act == "relu":
      out[...] = jnp.maximum(y, 0.0)
    else:
      out[...] = jnp.where(y > 0.0, y, jnp.exp(jnp.minimum(y, 0.0)) - 1.0)

  row_blk = pl.BlockSpec((_R, D), lambda i: (i, 0))
  cnt_blk = pl.BlockSpec((_R, CW), lambda i: (i, 0))
  full = pl.BlockSpec((D, D), lambda i: (0, 0))
  bias_blk = pl.BlockSpec((1, D), lambda i: (0, 0))
  return pl.pallas_call(
      body,
      grid=(N // _R,),
      in_specs=[row_blk, row_blk, cnt_blk, cnt_blk, row_blk, full, full,
                bias_blk],
      out_specs=row_blk,
      out_shape=jax.ShapeDtypeStruct((N, D), jnp.float32),
  )


_tc_relu = _make_tc_layer("relu")
_tc_elu = _make_tc_layer("elu")


def kernel(x, edge_index, edge_feats,
           Wl_0_0, bl_0_0, Wr_0_0, Wl_0_1, bl_0_1, Wr_0_1,
           Wl_1_0, bl_1_0, Wr_1_0, Wl_1_1, bl_1_1, Wr_1_1):
  src2 = edge_index[0].reshape(NW * NCHUNK, C)
  dst2 = edge_index[1].reshape(NW * NCHUNK, C)
  zrows = jnp.zeros((NP, D), jnp.float32)
  zcnt = jnp.zeros((NP, CW), jnp.float32)
  ones = jnp.ones((C, CW), jnp.float32)

  layers = [
      (Wl_0_0, bl_0_0, Wr_0_0, _tc_relu),
      (Wl_0_1, bl_0_1, Wr_0_1, _tc_elu),
      (Wl_1_0, bl_1_0, Wr_1_0, _tc_relu),
      (Wl_1_1, bl_1_1, Wr_1_1, _tc_elu),
  ]

  c = _sc_count(dst2, zcnt, ones)
  c0, c1 = c[0], c[1]
  h = x
  for wl, bias, wr, tc in layers:
    p = _sc_agg(h, src2, dst2, zrows)
    h = tc(p[0], p[1], c0, c1, h, wl, wr, bias.reshape(1, D))
  return h


# speedup vs baseline: 8.8230x; 1.2203x over previous
import jax
import jax.numpy as jnp
from jax import lax
from jax.experimental import pallas as pl
from jax.experimental.pallas import tpu as pltpu
from jax.experimental.pallas import tpu_sc as plsc

N = 10000
E = 320000
D = 128
NC = 2
NS = 16
NW = NC * NS
EW = E // NW
C = 125
NCHUNK = EW // C
G = 16
NG = NCHUNK // G
NP = 10112
RPT = NP // NS
CW = 128

_mesh = plsc.VectorSubcoreMesh(core_axis_name="c", subcore_axis_name="s")


def _sc_agg_body(h, srcs, dsts, zrows, out_p, srcv, dstv, rows0, rows1, acc,
                 sem):
  cid = lax.axis_index("c")
  sid = lax.axis_index("s")
  wid = cid * NS + sid
  pltpu.sync_copy(zrows.at[pl.ds(sid * RPT, RPT)], acc.at[pl.ds(sid * RPT, RPT)])
  plsc.subcore_barrier()
  rows = (rows0, rows1)

  def group(g, carry):
    pltpu.sync_copy(srcs.at[pl.ds(wid * NCHUNK + g * G, G)], srcv)
    pltpu.sync_copy(dsts.at[pl.ds(wid * NCHUNK + g * G, G)], dstv)
    cp = pltpu.async_copy(h.at[srcv.at[0]], rows[0], sem)
    for j in range(G):
      cp.wait()
      if j + 1 < G:
        cp = pltpu.async_copy(h.at[srcv.at[j + 1]], rows[(j + 1) % 2], sem)
      pltpu.sync_copy(rows[j % 2], acc.at[dstv.at[j]], add=True)
    return carry

  lax.fori_loop(0, NG, group, 0)
  plsc.subcore_barrier()
  pltpu.sync_copy(acc.at[pl.ds(sid * RPT, RPT)],
                  out_p.at[cid, pl.ds(sid * RPT, RPT)])


_sc_agg = pl.kernel(
    _sc_agg_body,
    out_type=jax.ShapeDtypeStruct((NC, NP, D), jnp.float32),
    mesh=_mesh,
    scratch_types=[
        pltpu.VMEM((G, C), jnp.int32),
        pltpu.VMEM((G, C), jnp.int32),
        pltpu.VMEM((C, D), jnp.float32),
        pltpu.VMEM((C, D), jnp.float32),
        pltpu.VMEM_SHARED((NP, D), jnp.float32),
        pltpu.SemaphoreType.DMA,
    ],
)


def _sc_count_body(dsts, zcnt, ones, out_c, dstv, onesv, cacc):
  cid = lax.axis_index("c")
  sid = lax.axis_index("s")
  wid = cid * NS + sid
  pltpu.sync_copy(ones, onesv)
  pltpu.sync_copy(zcnt.at[pl.ds(sid * RPT, RPT)],
                  cacc.at[pl.ds(sid * RPT, RPT)])
  plsc.subcore_barrier()

  def group(g, carry):
    pltpu.sync_copy(dsts.at[pl.ds(wid * NCHUNK + g * G, G)], dstv)

    def chunk(j, c2):
      pltpu.sync_copy(onesv, cacc.at[dstv.at[j]], add=True)
      return c2

    return lax.fori_loop(0, G, chunk, carry)

  lax.fori_loop(0, NG, group, 0)
  plsc.subcore_barrier()
  pltpu.sync_copy(cacc.at[pl.ds(sid * RPT, RPT)],
                  out_c.at[cid, pl.ds(sid * RPT, RPT)])


_sc_count = pl.kernel(
    _sc_count_body,
    out_type=jax.ShapeDtypeStruct((NC, NP, CW), jnp.float32),
    mesh=_mesh,
    scratch_types=[
        pltpu.VMEM((G, C), jnp.int32),
        pltpu.VMEM((C, CW), jnp.float32),
        pltpu.VMEM_SHARED((NP, CW), jnp.float32),
    ],
)

_R = 400


def _make_tc_layer(act):
  def body(p0, p1, c0, c1, h, wl, wr, bias, out):
    cnt = c0[:, 0:1] + c1[:, 0:1]
    inv = 1.0 / jnp.maximum(cnt, 1.0)
    agg = (p0[...] + p1[...]) * inv
    y = (jnp.dot(agg, wl[...], preferred_element_type=jnp.float32)
         + jnp.dot(h[...], wr[...], preferred_element_type=jnp.float32)
         + bias[...])
    if act == "relu":
      out[...] = jnp.maximum(y, 0.0)
    else:
      out[...] = jnp.where(y > 0.0, y, jnp.exp(jnp.minimum(y, 0.0)) - 1.0)

  row_blk = pl.BlockSpec((_R, D), lambda i: (i, 0))
  cnt_blk = pl.BlockSpec((_R, CW), lambda i: (i, 0))
  full = pl.BlockSpec((D, D), lambda i: (0, 0))
  bias_blk = pl.BlockSpec((1, D), lambda i: (0, 0))
  return pl.pallas_call(
      body,
      grid=(N // _R,),
      in_specs=[row_blk, row_blk, cnt_blk, cnt_blk, row_blk, full, full,
                bias_blk],
      out_specs=row_blk,
      out_shape=jax.ShapeDtypeStruct((N, D), jnp.float32),
  )


_tc_relu = _make_tc_layer("relu")
_tc_elu = _make_tc_layer("elu")


def kernel(x, edge_index, edge_feats,
           Wl_0_0, bl_0_0, Wr_0_0, Wl_0_1, bl_0_1, Wr_0_1,
           Wl_1_0, bl_1_0, Wr_1_0, Wl_1_1, bl_1_1, Wr_1_1):
  src2 = edge_index[0].reshape(NW * NCHUNK, C)
  dst2 = edge_index[1].reshape(NW * NCHUNK, C)
  zrows = jnp.zeros((NP, D), jnp.float32)
  zcnt = jnp.zeros((NP, CW), jnp.float32)
  ones = jnp.ones((C, CW), jnp.float32)

  layers = [
      (Wl_0_0, bl_0_0, Wr_0_0, _tc_relu),
      (Wl_0_1, bl_0_1, Wr_0_1, _tc_elu),
      (Wl_1_0, bl_1_0, Wr_1_0, _tc_relu),
      (Wl_1_1, bl_1_1, Wr_1_1, _tc_elu),
  ]

  c = _sc_count(dst2, zcnt, ones)
  c0, c1 = c[0], c[1]
  h = x
  for wl, bias, wr, tc in layers:
    p = _sc_agg(h, src2, dst2, zrows)
    h = tc(p[0], p[1], c0, c1, h, wl, wr, bias.reshape(1, D))
  return h
